# Initial kernel scaffold; baseline (speedup 1.0000x reference)
import jax
import jax.numpy as jnp
from jax.experimental import pallas as pl


def kernel(x, emb, n1_scale, n1_bias, W_dkv, W_dq, W_uk, W_uv, W_uq, W_kr, W_qr, W_o, n2_scale, n2_bias, sh_up, sh_down, r_up, r_down, router, nf_scale, nf_bias, proj):
    raise NotImplementedError("write your pallas kernel here")


# speedup vs baseline: 1.6234x; 1.6234x over previous
import functools
import math

import jax
import jax.numpy as jnp
from jax import lax
from jax.experimental import pallas as pl
from jax.experimental.pallas import tpu as pltpu
from jax.experimental.pallas import tpu_sc as plsc

H = 768
NH = 12
CD = 256
VOCAB = 32000
NE = 8
ED = 3072
DH = 64
DR = 32
UP = (DH - DR) * NH
SEQ = 2048

_SQRT2 = math.sqrt(2.0)


def _ln_m(x, scale, bias):
    mu = jnp.mean(x, axis=-1, keepdims=True)
    d = x - mu
    var = jnp.mean(d * d, axis=-1, keepdims=True)
    return d * lax.rsqrt(var + 1e-5) * scale + bias


def _gelu(x):
    return 0.5 * x * (1.0 + lax.erf(x / _SQRT2))


def _sc_gather(emb, idx):
    info = plsc.get_sparse_core_info()
    nw = info.num_cores * info.num_subcores
    n = idx.shape[0]
    bpw = n // nw
    mesh = plsc.VectorSubcoreMesh(core_axis_name="c", subcore_axis_name="s")

    @functools.partial(
        pl.kernel,
        mesh=mesh,
        out_type=jax.ShapeDtypeStruct((n, emb.shape[1]), jnp.float32),
        scratch_types=[
            pltpu.VMEM((bpw,), jnp.int32),
            pltpu.VMEM((bpw, emb.shape[1]), jnp.float32),
            pltpu.SemaphoreType.DMA,
        ],
    )
    def k(table_hbm, idx_hbm, out_hbm, idx_v, rows_v, sem):
        wid = lax.axis_index("s") * info.num_cores + lax.axis_index("c")
        base = wid * bpw
        pltpu.sync_copy(idx_hbm.at[pl.ds(base, bpw)], idx_v)
        pltpu.async_copy(table_hbm.at[idx_v], rows_v, sem).wait()
        pltpu.sync_copy(rows_v, out_hbm.at[pl.ds(base, bpw)])

    return k(emb, idx)


def _rope_full(x, cos_f, sin_f, lane_mod):
    x_m = jnp.roll(x, -8, axis=1)
    x_p = jnp.roll(x, 8, axis=1)
    rot_lo = x * cos_f - x_m * sin_f
    rot_hi = x * cos_f + x_p * sin_f
    return jnp.where(lane_mod < 8, rot_lo, jnp.where(lane_mod < 16, rot_hi, x))


def _attn_body(h_ref, n1s, n1b, wdkv, wdq, wuk, wuv, wuq, wkr, wqr, wo, a_ref):
    f32 = jnp.float32
    dot = lambda a, b: jnp.dot(a, b, preferred_element_type=f32)
    h = h_ref[...]
    hn = _ln_m(h, n1s[...], n1b[...])
    c_kv = dot(hn, wdkv[...])
    c_q = dot(hn, wdq[...])
    qc = dot(c_q, wuq[...])
    kc = dot(c_kv, wuk[...])
    v = dot(c_kv, wuv[...])
    kr = dot(hn, wkr[...])
    qr = dot(c_q, wqr[...])

    s = h.shape[0]
    lane = lax.broadcasted_iota(jnp.int32, (s, UP), 1)
    lane_mod = lane & 31
    freq = jnp.exp((lane_mod & 7).astype(f32) * (-math.log(10000.0) / 8.0))
    t = lax.broadcasted_iota(jnp.int32, (s, UP), 0).astype(f32) / 40.0
    ang = t * freq
    cos_f = jnp.cos(ang)
    sin_f = jnp.sin(ang)
    kr = _rope_full(kr, cos_f, sin_f, lane_mod)
    qr = _rope_full(qr, cos_f, sin_f, lane_mod)

    i_up = lax.broadcasted_iota(jnp.int32, (UP, H), 0)
    j_h = lax.broadcasted_iota(jnp.int32, (UP, H), 1)
    mask = ((i_up >> 5) == (j_h >> 6)).astype(f32)

    dot_t = lambda a, b: lax.dot_general(a, b, (((0,), (0,)), ((), ())),
                                         preferred_element_type=f32)
    mc = dot_t(qc, v) * mask
    mr = dot_t(qr, v) * mask
    out = (dot(kc, mc) + dot(kr, mr)) * 0.125
    a_ref[...] = dot(out, wo[...]) + h


def _attn_call(h, n1s, n1b, wdkv, wdq, wuk, wuv, wuq, wkr, wqr, wo):
    return pl.pallas_call(
        _attn_body,
        out_shape=jax.ShapeDtypeStruct((SEQ, H), jnp.float32),
    )(h, n1s, n1b, wdkv, wdq, wuk, wuv, wuq, wkr, wqr, wo)


def _moe_body(a_ref, h_ref, n2s, n2b, shup, shdown, rup, rdown, router, m_ref):
    f32 = jnp.float32
    bf16 = jnp.bfloat16
    dot = lambda a, b: jnp.dot(a, b, preferred_element_type=f32)
    e = pl.program_id(1)
    c = pl.program_id(2)
    a = a_ref[...]
    xn = _ln_m(a, n2s[...], n2b[...])
    logits = dot(xn, router[...])
    mx = jnp.max(logits, axis=-1, keepdims=True)
    ex = jnp.exp(logits - mx)
    p = ex / jnp.sum(ex, axis=-1, keepdims=True)
    m1 = jnp.max(p, axis=-1, keepdims=True)
    lane = lax.broadcasted_iota(jnp.int32, p.shape, 1)
    first_i = jnp.min(jnp.where(p == m1, lane, NE), axis=-1, keepdims=True)
    p_wo = jnp.where(lane == first_i, -1.0, p)
    m2 = jnp.max(p_wo, axis=-1, keepdims=True)
    w = jnp.where(p >= m2, p, 0.0)
    w_e = jnp.sum(w * (lane == e).astype(f32), axis=-1, keepdims=True)

    xnb = xn.astype(bf16)

    @pl.when((e == 0) & (c == 0))
    def _():
        m_ref[...] = xn + h_ref[...]

    @pl.when(e == 0)
    def _():
        m_ref[...] += dot(_gelu(dot(xnb, shup[...])).astype(bf16), shdown[...])

    contrib = dot(_gelu(dot(xnb, rup[0])).astype(bf16), rdown[0]) * w_e
    m_ref[...] += contrib


def _moe_call(a, h, n2s, n2b, shup, shdown, rup, rdown, router):
    bs = 512
    cd = ED // 2
    grid = (SEQ // bs, NE, ED // cd)
    return pl.pallas_call(
        _moe_body,
        grid=grid,
        in_specs=[
            pl.BlockSpec((bs, H), lambda s, e, c: (s, 0)),
            pl.BlockSpec((bs, H), lambda s, e, c: (s, 0)),
            pl.BlockSpec((1, H), lambda s, e, c: (0, 0)),
            pl.BlockSpec((1, H), lambda s, e, c: (0, 0)),
            pl.BlockSpec((H, cd), lambda s, e, c: (0, c)),
            pl.BlockSpec((cd, H), lambda s, e, c: (c, 0)),
            pl.BlockSpec((1, H, cd), lambda s, e, c: (e, 0, c)),
            pl.BlockSpec((1, cd, H), lambda s, e, c: (e, c, 0)),
            pl.BlockSpec((H, NE), lambda s, e, c: (0, 0)),
        ],
        out_specs=pl.BlockSpec((bs, H), lambda s, e, c: (s, 0)),
        out_shape=jax.ShapeDtypeStruct((SEQ, H), jnp.float32),
        compiler_params=pltpu.CompilerParams(
            dimension_semantics=("arbitrary", "arbitrary", "arbitrary")),
    )(a, h, n2s, n2b, shup, shdown, rup, rdown, router)


def _head_body(m_ref, nfs, nfb, proj_ref, o_ref):
    mn = _ln_m(m_ref[...], nfs[...], nfb[...]).astype(jnp.bfloat16)
    o_ref[...] = jnp.dot(mn, proj_ref[...], preferred_element_type=jnp.float32)


def _head_call(m, nfs, nfb, proj):
    vb = 1280
    grid = (VOCAB // vb,)
    return pl.pallas_call(
        _head_body,
        grid=grid,
        in_specs=[
            pl.BlockSpec((SEQ, H), lambda v: (0, 0)),
            pl.BlockSpec((1, H), lambda v: (0, 0)),
            pl.BlockSpec((1, H), lambda v: (0, 0)),
            pl.BlockSpec((H, vb), lambda v: (0, v)),
        ],
        out_specs=pl.BlockSpec((SEQ, vb), lambda v: (0, v)),
        out_shape=jax.ShapeDtypeStruct((SEQ, VOCAB), jnp.float32),
        compiler_params=pltpu.CompilerParams(
            dimension_semantics=("arbitrary",)),
    )(m, nfs, nfb, proj)


def kernel(x, emb, n1_scale, n1_bias, W_dkv, W_dq, W_uk, W_uv, W_uq, W_kr,
           W_qr, W_o, n2_scale, n2_bias, sh_up, sh_down, r_up, r_down,
           router, nf_scale, nf_bias, proj):
    b, s = x.shape
    idx = x.reshape(s)
    h = _sc_gather(emb, idx)
    n1s = n1_scale.reshape(1, H)
    n1b = n1_bias.reshape(1, H)
    a = _attn_call(h, n1s, n1b, W_dkv, W_dq, W_uk, W_uv, W_uq, W_kr, W_qr, W_o)
    bf16 = jnp.bfloat16
    m = _moe_call(a, h, n2_scale.reshape(1, H), n2_bias.reshape(1, H),
                  sh_up.astype(bf16), sh_down.astype(bf16),
                  r_up.astype(bf16), r_down.astype(bf16), router)
    out = _head_call(m, nf_scale.reshape(1, H), nf_bias.reshape(1, H),
                     proj.astype(bf16))
    return out.reshape(b, s, VOCAB)
